# baseline (speedup 1.0000x reference)
import jax
import jax.numpy as jnp
from jax.experimental import pallas as pl
from jax.experimental.pallas import tpu as pltpu

_B, _C, _H, _W = 4, 256, 64, 64
_K = 150
_KP = 152
_T = 64
_HW = _H * _W
_EPS = 1e-5
_NEG = -1e30
_BQ = 512

_PREC = jax.lax.Precision.DEFAULT


def _mm0(w, a):
    return jax.lax.dot_general(w, a, (((0,), (0,)), ((), ())),
                               preferred_element_type=jnp.float32,
                               precision=_PREC)


def _dot3(a, b16, dims):
    ahi = a.astype(jnp.bfloat16)
    r = a - ahi.astype(jnp.float32)
    amid = r.astype(jnp.bfloat16)
    alo = (r - amid.astype(jnp.float32)).astype(jnp.bfloat16)
    dot = lambda t: jax.lax.dot_general(t, b16, dims,
                                        preferred_element_type=jnp.float32,
                                        precision=_PREC)
    return dot(ahi) + (dot(amid) + dot(alo))


def _gather_kernel(x_ref, preds_ref, wq1_ref, wk1_ref, wv_ref,
                   yq_ref, yk_ref, yv_ref):
    l = preds_ref[0]
    s = jnp.max(l, axis=0, keepdims=True)
    kio = jax.lax.broadcasted_iota(jnp.int32, (_K, _HW), 0)
    seg = jnp.min(jnp.where(l == s, kio, _K), axis=0, keepdims=True)
    kio2 = jax.lax.broadcasted_iota(jnp.int32, (_KP, _HW), 0)
    m = kio2 == seg
    sb = jnp.broadcast_to(s, (_KP, _HW))
    seg_max = jnp.max(jnp.where(m, sb, _NEG), axis=1, keepdims=True)
    smax_p = jnp.max(jnp.where(m, jnp.broadcast_to(seg_max, (_KP, _HW)), _NEG),
                     axis=0, keepdims=True)
    e = jnp.exp(s - smax_p)
    mf = m.astype(jnp.float32)
    denom = jnp.sum(mf * e, axis=1, keepdims=True)
    denom_p = jnp.sum(jnp.where(m, jnp.broadcast_to(denom, (_KP, _HW)), 0.0),
                      axis=0, keepdims=True)
    wgt = e / denom_p
    x = x_ref[0]
    fw = x * wgt
    m16 = m.astype(jnp.bfloat16)
    ctx = _dot3(fw, m16, (((1,), (1,)), ((), ())))
    fsl = _dot3(ctx, m16, (((1,), (0,)), ((), ())))
    yq_ref[0] = _mm0(wq1_ref[...], x)
    yk_ref[0] = _mm0(wk1_ref[...], fsl)
    yv_ref[0] = _mm0(wv_ref[...], fsl)


def _stats_of(ref):
    ssum = jnp.zeros((_T, 1), jnp.float32)
    ssq = jnp.zeros((_T, 1), jnp.float32)
    for i in range(_B):
        y = ref[i]
        ssum = ssum + jnp.sum(y, axis=1, keepdims=True)
        ssq = ssq + jnp.sum(y * y, axis=1, keepdims=True)
    return ssum, ssq


def _bn_coefs(stats, g, b):
    ssum, ssq = stats
    n = float(_B * _HW)
    mean = ssum / n
    var = ssq / n - mean * mean
    inv = jax.lax.rsqrt(var + _EPS) * g
    return inv, b - mean * inv


def _bn_relu_to(src_ref, dst_ref, stats, g, b):
    inv, off = _bn_coefs(stats, g, b)
    for i in range(_B):
        dst_ref[i] = jnp.maximum(src_ref[i] * inv + off, 0.0)


def _bn_relu_cast_to(src_ref, dst_ref, stats, g, b, scale):
    inv, off = _bn_coefs(stats, g, b)
    for i in range(_B):
        a = jnp.maximum(src_ref[i] * inv + off, 0.0)
        dst_ref[i] = (a * scale).astype(jnp.bfloat16)


def _mm_layer(in_fn, w, out_ref):
    ssum = jnp.zeros((w.shape[1], 1), jnp.float32)
    ssq = jnp.zeros((w.shape[1], 1), jnp.float32)
    for i in range(_B):
        y = _mm0(w, in_fn(i))
        out_ref[i] = y
        ssum = ssum + jnp.sum(y, axis=1, keepdims=True)
        ssq = ssq + jnp.sum(y * y, axis=1, keepdims=True)
    return ssum, ssq


def _bn_relu_inplace(ref, stats, g, b):
    inv, off = _bn_coefs(stats, g, b)
    for i in range(_B):
        ref[i] = jnp.maximum(ref[i] * inv + off, 0.0)


def _proj_kernel(yq_ref, yk_ref, yv_ref, wq2_ref, gq1_ref, bq1_ref, gq2_ref,
                 bq2_ref, wk2_ref, gk1_ref, bk1_ref, gk2_ref, bk2_ref,
                 gv_ref, bv_ref, q_ref, k_ref, v_ref, sc_ref):
    _bn_relu_to(yq_ref, sc_ref, _stats_of(yq_ref), gq1_ref[...], bq1_ref[...])
    st = _mm_layer(lambda i: sc_ref[i], wq2_ref[...], sc_ref)
    _bn_relu_cast_to(sc_ref, q_ref, st, gq2_ref[...], bq2_ref[...], _T ** -0.5)
    _bn_relu_to(yk_ref, sc_ref, _stats_of(yk_ref), gk1_ref[...], bk1_ref[...])
    st = _mm_layer(lambda i: sc_ref[i], wk2_ref[...], sc_ref)
    _bn_relu_cast_to(sc_ref, k_ref, st, gk2_ref[...], bk2_ref[...], 1.0)
    _bn_relu_cast_to(yv_ref, v_ref, _stats_of(yv_ref), gv_ref[...], bv_ref[...],
                     1.0)


_NI = _HW // _BQ


def _attn_kernel(q_ref, k_ref, v_ref, o_ref, s_ref):
    i = pl.program_id(1)
    slot = i % 2
    prev = (i - 1) % 2
    sim = jax.lax.dot_general(q_ref[0], k_ref[0], (((0,), (0,)), ((), ())),
                              preferred_element_type=jnp.float32,
                              precision=_PREC)
    s_ref[pl.ds(slot, 1)] = sim[None]
    s = s_ref[pl.ds(prev, 1)][0]
    mx = jnp.max(s, axis=1, keepdims=True)
    p = jnp.exp(s - mx)
    denom = jnp.sum(p, axis=1, keepdims=True)
    p = (p / denom).astype(jnp.bfloat16)
    ctx = jax.lax.dot_general(v_ref[0], p, (((1,), (1,)), ((), ())),
                              preferred_element_type=jnp.float32,
                              precision=_PREC)
    o_ref[0, :, pl.ds(jnp.maximum(i - 1, 0) * _BQ, _BQ)] = ctx


def _out_kernel(c_ref, wo_ref, go_ref, bo_ref, out_ref):
    st = _mm_layer(lambda i: c_ref[i], wo_ref[...], out_ref)
    _bn_relu_inplace(out_ref, st, go_ref[...], bo_ref[...])


def kernel(x, preds, feats_il, Wq1, gq1, bq1, Wq2, gq2, bq2,
           Wk1, gk1, bk1, Wk2, gk2, bk2, Wv, gv, bv, Wo, go, bo):
    del feats_il
    xf = x.reshape(_B, _C, _HW)
    lg = preds.reshape(_B, _K, _HW)
    col = lambda v: v.reshape(-1, 1)

    full = lambda shp: pl.BlockSpec(shp, lambda *_: (0,) * len(shp))
    perb = lambda shp: pl.BlockSpec(shp, lambda b, *_: (b,) + (0,) * (len(shp) - 1))
    bthw = jax.ShapeDtypeStruct((_B, _T, _HW), jnp.float32)
    bthw16 = jax.ShapeDtypeStruct((_B, _T, _HW), jnp.bfloat16)

    yq, yk, yv = pl.pallas_call(
        _gather_kernel,
        grid=(_B,),
        in_specs=[perb((1, _C, _HW)), perb((1, _K, _HW)),
                  full((_C, _T)), full((_C, _T)), full((_C, _T))],
        out_specs=[perb((1, _T, _HW))] * 3,
        out_shape=[bthw] * 3,
    )(xf, lg, Wq1, Wk1, Wv)

    q, k, v = pl.pallas_call(
        _proj_kernel,
        in_specs=[full((_B, _T, _HW))] * 3 +
                 [full((_T, _T)), full((_T, 1)), full((_T, 1)), full((_T, 1)),
                  full((_T, 1)),
                  full((_T, _T)), full((_T, 1)), full((_T, 1)), full((_T, 1)),
                  full((_T, 1)), full((_T, 1)), full((_T, 1))],
        out_specs=[full((_B, _T, _HW))] * 3,
        out_shape=[bthw16] * 3,
        scratch_shapes=[pltpu.VMEM((_B, _T, _HW), jnp.float32)],
    )(yq, yk, yv, Wq2, col(gq1), col(bq1), col(gq2), col(bq2),
      Wk2, col(gk1), col(bk1), col(gk2), col(bk2), col(gv), col(bv))

    ctx = pl.pallas_call(
        _attn_kernel,
        grid=(_B, _NI + 1),
        in_specs=[pl.BlockSpec((1, _T, _BQ),
                               lambda b, i: (b, 0, jnp.minimum(i, _NI - 1))),
                  pl.BlockSpec((1, _T, _HW), lambda b, i: (b, 0, 0)),
                  pl.BlockSpec((1, _T, _HW), lambda b, i: (b, 0, 0))],
        out_specs=pl.BlockSpec((1, _T, _HW), lambda b, i: (b, 0, 0)),
        out_shape=bthw,
        scratch_shapes=[pltpu.VMEM((2, _BQ, _HW), jnp.float32)],
    )(q, k, v)

    out = pl.pallas_call(
        _out_kernel,
        in_specs=[full((_B, _T, _HW)), full((_T, _C)), full((_C, 1)), full((_C, 1))],
        out_specs=full((_B, _C, _HW)),
        out_shape=jax.ShapeDtypeStruct((_B, _C, _HW), jnp.float32),
    )(ctx, Wo, col(go), col(bo))

    return out.reshape(_B, _C, _H, _W)


# speedup vs baseline: 1.1983x; 1.1983x over previous
import jax
import jax.numpy as jnp
from jax.experimental import pallas as pl
from jax.experimental.pallas import tpu as pltpu

_B, _C, _H, _W = 4, 256, 64, 64
_K = 150
_KP = 152
_T = 64
_HW = _H * _W
_EPS = 1e-5
_NEG = -1e30
_BQ = 1024

_PREC = jax.lax.Precision.DEFAULT


def _mm0(w, a):
    return jax.lax.dot_general(w, a, (((0,), (0,)), ((), ())),
                               preferred_element_type=jnp.float32,
                               precision=_PREC)


def _dot3(a, b16, dims):
    ahi = a.astype(jnp.bfloat16)
    r = a - ahi.astype(jnp.float32)
    amid = r.astype(jnp.bfloat16)
    alo = (r - amid.astype(jnp.float32)).astype(jnp.bfloat16)
    dot = lambda t: jax.lax.dot_general(t, b16, dims,
                                        preferred_element_type=jnp.float32,
                                        precision=_PREC)
    return dot(ahi) + (dot(amid) + dot(alo))


def _gather_kernel(x_ref, preds_ref, wq1_ref, wk1_ref, wv_ref,
                   yq_ref, yk_ref, yv_ref):
    l = preds_ref[0]
    s = jnp.max(l, axis=0, keepdims=True)
    kio = jax.lax.broadcasted_iota(jnp.int32, (_K, _HW), 0)
    seg = jnp.min(jnp.where(l == s, kio, _K), axis=0, keepdims=True)
    kio2 = jax.lax.broadcasted_iota(jnp.int32, (_KP, _HW), 0)
    m = kio2 == seg
    sb = jnp.broadcast_to(s, (_KP, _HW))
    seg_max = jnp.max(jnp.where(m, sb, _NEG), axis=1, keepdims=True)
    smax_p = jnp.max(jnp.where(m, jnp.broadcast_to(seg_max, (_KP, _HW)), _NEG),
                     axis=0, keepdims=True)
    e = jnp.exp(s - smax_p)
    mf = m.astype(jnp.float32)
    denom = jnp.sum(mf * e, axis=1, keepdims=True)
    denom_p = jnp.sum(jnp.where(m, jnp.broadcast_to(denom, (_KP, _HW)), 0.0),
                      axis=0, keepdims=True)
    wgt = e / denom_p
    x = x_ref[0]
    fw = x * wgt
    m16 = m.astype(jnp.bfloat16)
    ctx = _dot3(fw, m16, (((1,), (1,)), ((), ())))
    fsl = _dot3(ctx, m16, (((1,), (0,)), ((), ())))
    yq_ref[0] = _mm0(wq1_ref[...], x)
    yk_ref[0] = _mm0(wk1_ref[...], fsl)
    yv_ref[0] = _mm0(wv_ref[...], fsl)


def _stats_of(ref):
    ssum = jnp.zeros((_T, 1), jnp.float32)
    ssq = jnp.zeros((_T, 1), jnp.float32)
    for i in range(_B):
        y = ref[i]
        ssum = ssum + jnp.sum(y, axis=1, keepdims=True)
        ssq = ssq + jnp.sum(y * y, axis=1, keepdims=True)
    return ssum, ssq


def _bn_coefs(stats, g, b):
    ssum, ssq = stats
    n = float(_B * _HW)
    mean = ssum / n
    var = ssq / n - mean * mean
    inv = jax.lax.rsqrt(var + _EPS) * g
    return inv, b - mean * inv


def _bn_relu_to(src_ref, dst_ref, stats, g, b):
    inv, off = _bn_coefs(stats, g, b)
    for i in range(_B):
        dst_ref[i] = jnp.maximum(src_ref[i] * inv + off, 0.0)


def _bn_relu_cast_to(src_ref, dst_ref, stats, g, b, scale):
    inv, off = _bn_coefs(stats, g, b)
    for i in range(_B):
        a = jnp.maximum(src_ref[i] * inv + off, 0.0)
        dst_ref[i] = (a * scale).astype(jnp.bfloat16)


def _mm_layer(in_fn, w, out_ref):
    ssum = jnp.zeros((w.shape[1], 1), jnp.float32)
    ssq = jnp.zeros((w.shape[1], 1), jnp.float32)
    for i in range(_B):
        y = _mm0(w, in_fn(i))
        out_ref[i] = y
        ssum = ssum + jnp.sum(y, axis=1, keepdims=True)
        ssq = ssq + jnp.sum(y * y, axis=1, keepdims=True)
    return ssum, ssq


def _bn_relu_inplace(ref, stats, g, b):
    inv, off = _bn_coefs(stats, g, b)
    for i in range(_B):
        ref[i] = jnp.maximum(ref[i] * inv + off, 0.0)


def _proj_kernel(yq_ref, yk_ref, yv_ref, wq2_ref, gq1_ref, bq1_ref, gq2_ref,
                 bq2_ref, wk2_ref, gk1_ref, bk1_ref, gk2_ref, bk2_ref,
                 gv_ref, bv_ref, q_ref, k_ref, v_ref, sc_ref):
    _bn_relu_to(yq_ref, sc_ref, _stats_of(yq_ref), gq1_ref[...], bq1_ref[...])
    st = _mm_layer(lambda i: sc_ref[i], wq2_ref[...], sc_ref)
    _bn_relu_cast_to(sc_ref, q_ref, st, gq2_ref[...], bq2_ref[...], _T ** -0.5)
    _bn_relu_to(yk_ref, sc_ref, _stats_of(yk_ref), gk1_ref[...], bk1_ref[...])
    st = _mm_layer(lambda i: sc_ref[i], wk2_ref[...], sc_ref)
    _bn_relu_cast_to(sc_ref, k_ref, st, gk2_ref[...], bk2_ref[...], 1.0)
    _bn_relu_cast_to(yv_ref, v_ref, _stats_of(yv_ref), gv_ref[...], bv_ref[...],
                     1.0)


_NI = _HW // _BQ


def _attn_kernel(q_ref, k_ref, v_ref, o_ref):
    s = jax.lax.dot_general(q_ref[0], k_ref[0], (((0,), (0,)), ((), ())),
                            preferred_element_type=jnp.float32,
                            precision=_PREC)
    mx = jnp.max(s, axis=1, keepdims=True)
    p = jnp.exp(s - mx)
    denom = jnp.sum(p, axis=1, keepdims=True)
    p = (p / denom).astype(jnp.bfloat16)
    o_ref[0] = jax.lax.dot_general(v_ref[0], p, (((1,), (1,)), ((), ())),
                                   preferred_element_type=jnp.float32,
                                   precision=_PREC)


def _out_kernel(c_ref, wo_ref, go_ref, bo_ref, out_ref):
    st = _mm_layer(lambda i: c_ref[i], wo_ref[...], out_ref)
    _bn_relu_inplace(out_ref, st, go_ref[...], bo_ref[...])


def kernel(x, preds, feats_il, Wq1, gq1, bq1, Wq2, gq2, bq2,
           Wk1, gk1, bk1, Wk2, gk2, bk2, Wv, gv, bv, Wo, go, bo):
    del feats_il
    xf = x.reshape(_B, _C, _HW)
    lg = preds.reshape(_B, _K, _HW)
    col = lambda v: v.reshape(-1, 1)

    full = lambda shp: pl.BlockSpec(shp, lambda *_: (0,) * len(shp))
    perb = lambda shp: pl.BlockSpec(shp, lambda b, *_: (b,) + (0,) * (len(shp) - 1))
    bthw = jax.ShapeDtypeStruct((_B, _T, _HW), jnp.float32)
    bthw16 = jax.ShapeDtypeStruct((_B, _T, _HW), jnp.bfloat16)

    yq, yk, yv = pl.pallas_call(
        _gather_kernel,
        grid=(_B,),
        in_specs=[perb((1, _C, _HW)), perb((1, _K, _HW)),
                  full((_C, _T)), full((_C, _T)), full((_C, _T))],
        out_specs=[perb((1, _T, _HW))] * 3,
        out_shape=[bthw] * 3,
    )(xf, lg, Wq1, Wk1, Wv)

    q, k, v = pl.pallas_call(
        _proj_kernel,
        in_specs=[full((_B, _T, _HW))] * 3 +
                 [full((_T, _T)), full((_T, 1)), full((_T, 1)), full((_T, 1)),
                  full((_T, 1)),
                  full((_T, _T)), full((_T, 1)), full((_T, 1)), full((_T, 1)),
                  full((_T, 1)), full((_T, 1)), full((_T, 1))],
        out_specs=[full((_B, _T, _HW))] * 3,
        out_shape=[bthw16] * 3,
        scratch_shapes=[pltpu.VMEM((_B, _T, _HW), jnp.float32)],
    )(yq, yk, yv, Wq2, col(gq1), col(bq1), col(gq2), col(bq2),
      Wk2, col(gk1), col(bk1), col(gk2), col(bk2), col(gv), col(bv))

    ctx = pl.pallas_call(
        _attn_kernel,
        grid=(_B, _NI),
        in_specs=[pl.BlockSpec((1, _T, _BQ), lambda b, i: (b, 0, i)),
                  pl.BlockSpec((1, _T, _HW), lambda b, i: (b, 0, 0)),
                  pl.BlockSpec((1, _T, _HW), lambda b, i: (b, 0, 0))],
        out_specs=pl.BlockSpec((1, _T, _BQ), lambda b, i: (b, 0, i)),
        out_shape=bthw,
    )(q, k, v)

    out = pl.pallas_call(
        _out_kernel,
        in_specs=[full((_B, _T, _HW)), full((_T, _C)), full((_C, 1)), full((_C, 1))],
        out_specs=full((_B, _C, _HW)),
        out_shape=jax.ShapeDtypeStruct((_B, _C, _HW), jnp.float32),
    )(ctx, Wo, col(go), col(bo))

    return out.reshape(_B, _C, _H, _W)


# speedup vs baseline: 1.2352x; 1.0308x over previous
import jax
import jax.numpy as jnp
from jax.experimental import pallas as pl
from jax.experimental.pallas import tpu as pltpu

_B, _C, _H, _W = 4, 256, 64, 64
_K = 150
_KP = 152
_T = 64
_HW = _H * _W
_EPS = 1e-5
_NEG = -1e30
_BQ = 1024

_PREC = jax.lax.Precision.DEFAULT


def _mm0(w, a):
    return jax.lax.dot_general(w, a, (((0,), (0,)), ((), ())),
                               preferred_element_type=jnp.float32,
                               precision=_PREC)


def _dot3(a, b16, dims):
    ahi = a.astype(jnp.bfloat16)
    r = a - ahi.astype(jnp.float32)
    amid = r.astype(jnp.bfloat16)
    alo = (r - amid.astype(jnp.float32)).astype(jnp.bfloat16)
    dot = lambda t: jax.lax.dot_general(t, b16, dims,
                                        preferred_element_type=jnp.float32,
                                        precision=_PREC)
    return dot(ahi) + (dot(amid) + dot(alo))


def _gather_kernel(x_ref, preds_ref, wq1_ref, wk1_ref, wv_ref,
                   yq_ref, yk_ref, yv_ref):
    l = preds_ref[0]
    s = jnp.max(l, axis=0, keepdims=True)
    kio = jax.lax.broadcasted_iota(jnp.int32, (_K, _HW), 0)
    seg = jnp.min(jnp.where(l == s, kio, _K), axis=0, keepdims=True)
    kio2 = jax.lax.broadcasted_iota(jnp.int32, (_KP, _HW), 0)
    m = kio2 == seg
    sb = jnp.broadcast_to(s, (_KP, _HW))
    seg_max = jnp.max(jnp.where(m, sb, _NEG), axis=1, keepdims=True)
    smax_p = jnp.max(jnp.where(m, jnp.broadcast_to(seg_max, (_KP, _HW)), _NEG),
                     axis=0, keepdims=True)
    e = jnp.exp(s - smax_p)
    mf = m.astype(jnp.float32)
    denom = jnp.sum(mf * e, axis=1, keepdims=True)
    denom_p = jnp.sum(jnp.where(m, jnp.broadcast_to(denom, (_KP, _HW)), 0.0),
                      axis=0, keepdims=True)
    wgt = e / denom_p
    x = x_ref[0]
    fw = x * wgt
    m16 = m.astype(jnp.bfloat16)
    ctx = _dot3(fw, m16, (((1,), (1,)), ((), ())))
    yq_ref[0] = _mm0(wq1_ref[...], x)
    ck = _mm0(wk1_ref[...], ctx)
    cv = _mm0(wv_ref[...], ctx)
    yk_ref[0] = _dot3(ck, m16, (((1,), (0,)), ((), ())))
    yv_ref[0] = _dot3(cv, m16, (((1,), (0,)), ((), ())))


def _stats_of(ref):
    ssum = jnp.zeros((_T, 1), jnp.float32)
    ssq = jnp.zeros((_T, 1), jnp.float32)
    for i in range(_B):
        y = ref[i]
        ssum = ssum + jnp.sum(y, axis=1, keepdims=True)
        ssq = ssq + jnp.sum(y * y, axis=1, keepdims=True)
    return ssum, ssq


def _bn_coefs(stats, g, b):
    ssum, ssq = stats
    n = float(_B * _HW)
    mean = ssum / n
    var = ssq / n - mean * mean
    inv = jax.lax.rsqrt(var + _EPS) * g
    return inv, b - mean * inv


def _bn_relu_to(src_ref, dst_ref, stats, g, b):
    inv, off = _bn_coefs(stats, g, b)
    for i in range(_B):
        dst_ref[i] = jnp.maximum(src_ref[i] * inv + off, 0.0)


def _bn_relu_cast_to(src_ref, dst_ref, stats, g, b, scale):
    inv, off = _bn_coefs(stats, g, b)
    for i in range(_B):
        a = jnp.maximum(src_ref[i] * inv + off, 0.0)
        dst_ref[i] = (a * scale).astype(jnp.bfloat16)


def _mm_layer(in_fn, w, out_ref):
    ssum = jnp.zeros((w.shape[1], 1), jnp.float32)
    ssq = jnp.zeros((w.shape[1], 1), jnp.float32)
    for i in range(_B):
        y = _mm0(w, in_fn(i))
        out_ref[i] = y
        ssum = ssum + jnp.sum(y, axis=1, keepdims=True)
        ssq = ssq + jnp.sum(y * y, axis=1, keepdims=True)
    return ssum, ssq


def _bn_relu_inplace(ref, stats, g, b):
    inv, off = _bn_coefs(stats, g, b)
    for i in range(_B):
        ref[i] = jnp.maximum(ref[i] * inv + off, 0.0)


def _proj_kernel(yq_ref, yk_ref, yv_ref, wq2_ref, gq1_ref, bq1_ref, gq2_ref,
                 bq2_ref, wk2_ref, gk1_ref, bk1_ref, gk2_ref, bk2_ref,
                 gv_ref, bv_ref, q_ref, k_ref, v_ref, sc_ref):
    _bn_relu_to(yq_ref, sc_ref, _stats_of(yq_ref), gq1_ref[...], bq1_ref[...])
    st = _mm_layer(lambda i: sc_ref[i], wq2_ref[...], sc_ref)
    _bn_relu_cast_to(sc_ref, q_ref, st, gq2_ref[...], bq2_ref[...], _T ** -0.5)
    _bn_relu_to(yk_ref, sc_ref, _stats_of(yk_ref), gk1_ref[...], bk1_ref[...])
    st = _mm_layer(lambda i: sc_ref[i], wk2_ref[...], sc_ref)
    _bn_relu_cast_to(sc_ref, k_ref, st, gk2_ref[...], bk2_ref[...], 1.0)
    _bn_relu_cast_to(yv_ref, v_ref, _stats_of(yv_ref), gv_ref[...], bv_ref[...],
                     1.0)


_NI = _HW // _BQ


def _attn_kernel(q_ref, k_ref, v_ref, o_ref):
    s = jax.lax.dot_general(q_ref[0], k_ref[0], (((0,), (0,)), ((), ())),
                            preferred_element_type=jnp.float32,
                            precision=_PREC)
    mx = jnp.max(s, axis=1, keepdims=True)
    p = jnp.exp(s - mx)
    denom = jnp.sum(p, axis=1, keepdims=True)
    p = (p / denom).astype(jnp.bfloat16)
    o_ref[0] = jax.lax.dot_general(v_ref[0], p, (((1,), (1,)), ((), ())),
                                   preferred_element_type=jnp.float32,
                                   precision=_PREC)


def _out_kernel(c_ref, wo_ref, go_ref, bo_ref, out_ref):
    st = _mm_layer(lambda i: c_ref[i], wo_ref[...], out_ref)
    _bn_relu_inplace(out_ref, st, go_ref[...], bo_ref[...])


def kernel(x, preds, feats_il, Wq1, gq1, bq1, Wq2, gq2, bq2,
           Wk1, gk1, bk1, Wk2, gk2, bk2, Wv, gv, bv, Wo, go, bo):
    del feats_il
    xf = x.reshape(_B, _C, _HW)
    lg = preds.reshape(_B, _K, _HW)
    col = lambda v: v.reshape(-1, 1)

    full = lambda shp: pl.BlockSpec(shp, lambda *_: (0,) * len(shp))
    perb = lambda shp: pl.BlockSpec(shp, lambda b, *_: (b,) + (0,) * (len(shp) - 1))
    bthw = jax.ShapeDtypeStruct((_B, _T, _HW), jnp.float32)
    bthw16 = jax.ShapeDtypeStruct((_B, _T, _HW), jnp.bfloat16)

    yq, yk, yv = pl.pallas_call(
        _gather_kernel,
        grid=(_B,),
        in_specs=[perb((1, _C, _HW)), perb((1, _K, _HW)),
                  full((_C, _T)), full((_C, _T)), full((_C, _T))],
        out_specs=[perb((1, _T, _HW))] * 3,
        out_shape=[bthw] * 3,
    )(xf, lg, Wq1, Wk1, Wv)

    q, k, v = pl.pallas_call(
        _proj_kernel,
        in_specs=[full((_B, _T, _HW))] * 3 +
                 [full((_T, _T)), full((_T, 1)), full((_T, 1)), full((_T, 1)),
                  full((_T, 1)),
                  full((_T, _T)), full((_T, 1)), full((_T, 1)), full((_T, 1)),
                  full((_T, 1)), full((_T, 1)), full((_T, 1))],
        out_specs=[full((_B, _T, _HW))] * 3,
        out_shape=[bthw16] * 3,
        scratch_shapes=[pltpu.VMEM((_B, _T, _HW), jnp.float32)],
    )(yq, yk, yv, Wq2, col(gq1), col(bq1), col(gq2), col(bq2),
      Wk2, col(gk1), col(bk1), col(gk2), col(bk2), col(gv), col(bv))

    ctx = pl.pallas_call(
        _attn_kernel,
        grid=(_B, _NI),
        in_specs=[pl.BlockSpec((1, _T, _BQ), lambda b, i: (b, 0, i)),
                  pl.BlockSpec((1, _T, _HW), lambda b, i: (b, 0, 0)),
                  pl.BlockSpec((1, _T, _HW), lambda b, i: (b, 0, 0))],
        out_specs=pl.BlockSpec((1, _T, _BQ), lambda b, i: (b, 0, i)),
        out_shape=bthw,
    )(q, k, v)

    out = pl.pallas_call(
        _out_kernel,
        in_specs=[full((_B, _T, _HW)), full((_T, _C)), full((_C, 1)), full((_C, 1))],
        out_specs=full((_B, _C, _HW)),
        out_shape=jax.ShapeDtypeStruct((_B, _C, _HW), jnp.float32),
    )(ctx, Wo, col(go), col(bo))

    return out.reshape(_B, _C, _H, _W)


# speedup vs baseline: 1.2359x; 1.0006x over previous
import jax
import jax.numpy as jnp
from jax.experimental import pallas as pl
from jax.experimental.pallas import tpu as pltpu

_B, _C, _H, _W = 4, 256, 64, 64
_K = 150
_KP = 152
_T = 64
_HW = _H * _W
_EPS = 1e-5
_NEG = -1e30
_BQ = 1024

_PREC = jax.lax.Precision.DEFAULT


def _mm0(w, a):
    return jax.lax.dot_general(w, a, (((0,), (0,)), ((), ())),
                               preferred_element_type=jnp.float32,
                               precision=_PREC)


def _dot3(a, b16, dims):
    ahi = a.astype(jnp.bfloat16)
    r = a - ahi.astype(jnp.float32)
    amid = r.astype(jnp.bfloat16)
    alo = (r - amid.astype(jnp.float32)).astype(jnp.bfloat16)
    dot = lambda t: jax.lax.dot_general(t, b16, dims,
                                        preferred_element_type=jnp.float32,
                                        precision=_PREC)
    return dot(ahi) + (dot(amid) + dot(alo))


def _gather_kernel(x_ref, preds_ref, wq1_ref, wk1_ref, wv_ref,
                   yq_ref, yk_ref, yv_ref):
    l = preds_ref[0]
    s = jnp.max(l, axis=0, keepdims=True)
    kio = jax.lax.broadcasted_iota(jnp.int32, (_K, _HW), 0)
    seg = jnp.min(jnp.where(l == s, kio, _K), axis=0, keepdims=True)
    kio2 = jax.lax.broadcasted_iota(jnp.int32, (_KP, _HW), 0)
    m = kio2 == seg
    sb = jnp.broadcast_to(s, (_KP, _HW))
    seg_max = jnp.max(jnp.where(m, sb, _NEG), axis=1, keepdims=True)
    smax_p = jnp.max(jnp.where(m, jnp.broadcast_to(seg_max, (_KP, _HW)), _NEG),
                     axis=0, keepdims=True)
    e = jnp.exp(s - smax_p)
    mf = m.astype(jnp.float32)
    denom = jnp.sum(mf * e, axis=1, keepdims=True)
    denom_p = jnp.sum(jnp.where(m, jnp.broadcast_to(denom, (_KP, _HW)), 0.0),
                      axis=0, keepdims=True)
    wgt = e / denom_p
    x = x_ref[0]
    fw = x * wgt
    m16 = m.astype(jnp.bfloat16)
    ctx = _dot3(fw, m16, (((1,), (1,)), ((), ())))
    yq_ref[0] = _mm0(wq1_ref[...], x)
    ck = _mm0(wk1_ref[...], ctx)
    cv = _mm0(wv_ref[...], ctx)
    yk_ref[0] = _dot3(ck, m16, (((1,), (0,)), ((), ())))
    yv_ref[0] = _dot3(cv, m16, (((1,), (0,)), ((), ())))


def _stats_of(ref):
    ssum = jnp.zeros((_T, 1), jnp.float32)
    ssq = jnp.zeros((_T, 1), jnp.float32)
    for i in range(_B):
        y = ref[i]
        ssum = ssum + jnp.sum(y, axis=1, keepdims=True)
        ssq = ssq + jnp.sum(y * y, axis=1, keepdims=True)
    return ssum, ssq


def _bn_coefs(stats, g, b):
    ssum, ssq = stats
    n = float(_B * _HW)
    mean = ssum / n
    var = ssq / n - mean * mean
    inv = jax.lax.rsqrt(var + _EPS) * g
    return inv, b - mean * inv


def _bn_relu_to(src_ref, dst_ref, stats, g, b):
    inv, off = _bn_coefs(stats, g, b)
    for i in range(_B):
        dst_ref[i] = jnp.maximum(src_ref[i] * inv + off, 0.0)


def _bn_relu_cast_to(src_ref, dst_ref, stats, g, b, scale):
    inv, off = _bn_coefs(stats, g, b)
    for i in range(_B):
        a = jnp.maximum(src_ref[i] * inv + off, 0.0)
        dst_ref[i] = (a * scale).astype(jnp.bfloat16)


def _mm_layer(in_fn, w, out_ref):
    ssum = jnp.zeros((w.shape[1], 1), jnp.float32)
    ssq = jnp.zeros((w.shape[1], 1), jnp.float32)
    for i in range(_B):
        y = _mm0(w, in_fn(i))
        out_ref[i] = y
        ssum = ssum + jnp.sum(y, axis=1, keepdims=True)
        ssq = ssq + jnp.sum(y * y, axis=1, keepdims=True)
    return ssum, ssq


def _proj_kernel(yq_ref, yk_ref, yv_ref, wq2_ref, gq1_ref, bq1_ref, gq2_ref,
                 bq2_ref, wk2_ref, gk1_ref, bk1_ref, gk2_ref, bk2_ref,
                 gv_ref, bv_ref, q_ref, k_ref, v_ref, sa_ref, sb_ref):
    _bn_relu_to(yq_ref, sa_ref, _stats_of(yq_ref), gq1_ref[...], bq1_ref[...])
    st = _mm_layer(lambda i: sa_ref[i], wq2_ref[...], sb_ref)
    _bn_relu_cast_to(sb_ref, q_ref, st, gq2_ref[...], bq2_ref[...], _T ** -0.5)
    _bn_relu_to(yk_ref, sa_ref, _stats_of(yk_ref), gk1_ref[...], bk1_ref[...])
    st = _mm_layer(lambda i: sa_ref[i], wk2_ref[...], sb_ref)
    _bn_relu_cast_to(sb_ref, k_ref, st, gk2_ref[...], bk2_ref[...], 1.0)
    _bn_relu_cast_to(yv_ref, v_ref, _stats_of(yv_ref), gv_ref[...], bv_ref[...],
                     1.0)


_NI = _HW // _BQ


def _attn_kernel(q_ref, k_ref, v_ref, o_ref):
    s = jax.lax.dot_general(q_ref[0], k_ref[0], (((0,), (0,)), ((), ())),
                            preferred_element_type=jnp.float32,
                            precision=_PREC)
    mx = jnp.max(s, axis=1, keepdims=True)
    p = jnp.exp(s - mx)
    denom = jnp.sum(p, axis=1, keepdims=True)
    p = (p / denom).astype(jnp.bfloat16)
    o_ref[0] = jax.lax.dot_general(v_ref[0], p, (((1,), (1,)), ((), ())),
                                   preferred_element_type=jnp.float32,
                                   precision=_PREC)


def _out_kernel(c_ref, wo_ref, go_ref, bo_ref, out_ref, sc_ref):
    st = _mm_layer(lambda i: c_ref[i], wo_ref[...], sc_ref)
    _bn_relu_to(sc_ref, out_ref, st, go_ref[...], bo_ref[...])


def kernel(x, preds, feats_il, Wq1, gq1, bq1, Wq2, gq2, bq2,
           Wk1, gk1, bk1, Wk2, gk2, bk2, Wv, gv, bv, Wo, go, bo):
    del feats_il
    xf = x.reshape(_B, _C, _HW)
    lg = preds.reshape(_B, _K, _HW)
    col = lambda v: v.reshape(-1, 1)

    full = lambda shp: pl.BlockSpec(shp, lambda *_: (0,) * len(shp))
    perb = lambda shp: pl.BlockSpec(shp, lambda b, *_: (b,) + (0,) * (len(shp) - 1))
    bthw = jax.ShapeDtypeStruct((_B, _T, _HW), jnp.float32)
    bthw16 = jax.ShapeDtypeStruct((_B, _T, _HW), jnp.bfloat16)

    yq, yk, yv = pl.pallas_call(
        _gather_kernel,
        grid=(_B,),
        in_specs=[perb((1, _C, _HW)), perb((1, _K, _HW)),
                  full((_C, _T)), full((_C, _T)), full((_C, _T))],
        out_specs=[perb((1, _T, _HW))] * 3,
        out_shape=[bthw] * 3,
    )(xf, lg, Wq1, Wk1, Wv)

    q, k, v = pl.pallas_call(
        _proj_kernel,
        in_specs=[full((_B, _T, _HW))] * 3 +
                 [full((_T, _T)), full((_T, 1)), full((_T, 1)), full((_T, 1)),
                  full((_T, 1)),
                  full((_T, _T)), full((_T, 1)), full((_T, 1)), full((_T, 1)),
                  full((_T, 1)), full((_T, 1)), full((_T, 1))],
        out_specs=[full((_B, _T, _HW))] * 3,
        out_shape=[bthw16] * 3,
        scratch_shapes=[pltpu.VMEM((_B, _T, _HW), jnp.float32),
                        pltpu.VMEM((_B, _T, _HW), jnp.float32)],
    )(yq, yk, yv, Wq2, col(gq1), col(bq1), col(gq2), col(bq2),
      Wk2, col(gk1), col(bk1), col(gk2), col(bk2), col(gv), col(bv))

    ctx = pl.pallas_call(
        _attn_kernel,
        grid=(_B, _NI),
        in_specs=[pl.BlockSpec((1, _T, _BQ), lambda b, i: (b, 0, i)),
                  pl.BlockSpec((1, _T, _HW), lambda b, i: (b, 0, 0)),
                  pl.BlockSpec((1, _T, _HW), lambda b, i: (b, 0, 0))],
        out_specs=pl.BlockSpec((1, _T, _BQ), lambda b, i: (b, 0, i)),
        out_shape=bthw,
    )(q, k, v)

    out = pl.pallas_call(
        _out_kernel,
        in_specs=[full((_B, _T, _HW)), full((_T, _C)), full((_C, 1)), full((_C, 1))],
        out_specs=full((_B, _C, _HW)),
        out_shape=jax.ShapeDtypeStruct((_B, _C, _HW), jnp.float32),
        scratch_shapes=[pltpu.VMEM((_B, _C, _HW), jnp.float32)],
    )(ctx, Wo, col(go), col(bo))

    return out.reshape(_B, _C, _H, _W)
